# baseline (speedup 1.0000x reference)
import functools

import jax
import jax.numpy as jnp
from jax import lax
from jax.experimental import pallas as pl
from jax.experimental.pallas import tpu as pltpu
from jax.experimental.pallas import tpu_sc as plsc

N_NODES = 10000
D_FEAT = 128
ROWS = 10112
N_EDGES = 320000
NC, NS = 2, 16
NW = NC * NS
K = 80
CH = N_EDGES // K
NCH0 = 130
NCH1 = 120
STRIPE = ROWS // NS

IN_DIM = 2 * D_FEAT
H2 = 192
EMB = 128


@functools.cache
def _make_sc_agg():
    mesh = plsc.VectorSubcoreMesh(
        core_axis_name="c", subcore_axis_name="s",
        num_cores=NC, num_subcores=NS)

    @functools.partial(
        pl.kernel,
        out_type=jax.ShapeDtypeStruct((NC, ROWS, D_FEAT), jnp.float32),
        mesh=mesh,
        scratch_types=[
            pltpu.VMEM((NCH0, K), jnp.int32),
            pltpu.VMEM((NCH0, K), jnp.int32),
            pltpu.VMEM((K, D_FEAT), jnp.float32),
            pltpu.VMEM((K, D_FEAT), jnp.float32),
            pltpu.VMEM_SHARED((ROWS, D_FEAT), jnp.float32),
            pltpu.SemaphoreType.DMA,
            pltpu.SemaphoreType.DMA,
            pltpu.SemaphoreType.DMA,
            pltpu.SemaphoreType.DMA,
        ],
        compiler_params=pltpu.CompilerParams(use_tc_tiling_on_sc=False,
                                             needs_layout_passes=False),
    )
    def sc_agg(x_hbm, epk_hbm, parts_out,
               sidx, didx, buf0, buf1, acc, sem0, sem1, ssm0, ssm1):
        c = lax.axis_index("c")
        s = lax.axis_index("s")
        zeros = jnp.zeros((16,), jnp.float32)
        ones = jnp.ones((16,), jnp.float32)
        zeros_i = jnp.zeros((16,), jnp.int32)
        iota16 = lax.broadcasted_iota(jnp.int32, (16,), 0)

        def set_count_col(buf):
            for g in range(K // 16):
                plsc.store_scatter(buf, [iota16 + (g * 16), zeros_i], ones)

        def _zrow(i, _):
            for g in range(D_FEAT // 16):
                buf0[i, pl.ds(g * 16, 16)] = zeros
            return _
        lax.fori_loop(0, K, _zrow, None)

        for kk in range(STRIPE // K):
            pltpu.sync_copy(buf0, acc.at[pl.ds(s * STRIPE + kk * K, K)])
        rem = STRIPE % K
        if rem:
            pltpu.sync_copy(buf0.at[pl.ds(0, rem)],
                            acc.at[pl.ds(s * STRIPE + (STRIPE // K) * K, rem)])
        plsc.subcore_barrier()

        def run_range(row0, nch):
            pltpu.sync_copy(epk_hbm.at[0, pl.ds(row0, nch)],
                            sidx.at[pl.ds(0, nch)])
            pltpu.sync_copy(epk_hbm.at[1, pl.ds(row0, nch)],
                            didx.at[pl.ds(0, nch)])
            pltpu.async_copy(x_hbm.at[sidx.at[0]], buf0, sem0)
            pltpu.async_copy(x_hbm.at[sidx.at[1]], buf1, sem1)

            def body(i, _):
                j = 2 * i
                pltpu.make_async_copy(x_hbm.at[sidx.at[j]], buf0,
                                      sem0).wait()
                set_count_col(buf0)
                pltpu.async_copy(buf0, acc.at[didx.at[j]], ssm0, add=True)
                pltpu.make_async_copy(x_hbm.at[sidx.at[j + 1]], buf1,
                                      sem1).wait()
                set_count_col(buf1)
                pltpu.async_copy(buf1, acc.at[didx.at[j + 1]], ssm1, add=True)

                @pl.when(j + 2 < nch)
                def _():
                    pltpu.make_async_copy(buf0, acc.at[didx.at[j]],
                                          ssm0).wait()
                    pltpu.async_copy(x_hbm.at[sidx.at[j + 2]], buf0, sem0)
                    pltpu.make_async_copy(buf1, acc.at[didx.at[j + 1]],
                                          ssm1).wait()
                    pltpu.async_copy(x_hbm.at[sidx.at[j + 3]], buf1, sem1)
                return _

            lax.fori_loop(0, nch // 2, body, None)
            pltpu.make_async_copy(buf0, acc.at[didx.at[0]], ssm0).wait()
            pltpu.make_async_copy(buf1, acc.at[didx.at[1]], ssm1).wait()

        @pl.when(c == 0)
        def _():
            run_range(s * NCH0, NCH0)

        @pl.when(c == 1)
        def _():
            run_range(NS * NCH0 + s * NCH1, NCH1)

        plsc.subcore_barrier()
        pltpu.sync_copy(acc.at[pl.ds(s * STRIPE, STRIPE)],
                        parts_out.at[c, pl.ds(s * STRIPE, STRIPE)])

    return sc_agg


def _tc_dense_body(x_ref, parts_ref, w1_ref, b1_ref, w2_ref, b2_ref,
                   w3_ref, b3_ref, w4_ref, b4_ref, enc_ref, dec_ref):
    xs = x_ref[...]
    p = parts_ref[0] + parts_ref[1]
    cnt = p[:, 0:1]
    agg = p / jnp.maximum(cnt, 1.0)
    col = lax.broadcasted_iota(jnp.int32, xs.shape, 1)
    xz = jnp.where(col == 0, 0.0, xs)
    aggz = jnp.where(col == 0, 0.0, agg)
    w1 = w1_ref[...]
    h = jnp.maximum(
        jnp.dot(xz, w1[:D_FEAT], preferred_element_type=jnp.float32)
        + jnp.dot(aggz, w1[D_FEAT:], preferred_element_type=jnp.float32)
        + b1_ref[...], 0.0)
    enc = jnp.dot(h, w2_ref[...], preferred_element_type=jnp.float32) + b2_ref[...]
    enc_ref[...] = enc
    h2 = jnp.maximum(
        jnp.dot(enc, w3_ref[...], preferred_element_type=jnp.float32)
        + b3_ref[...], 0.0)
    dec_ref[...] = (jnp.dot(h2, w4_ref[...], preferred_element_type=jnp.float32)
                    + b4_ref[...])


_TC_R = 2528


def _tc_dense(xp, parts, W_enc1, b_enc1, W_enc3, b_enc3,
              W_dec1, b_dec1, W_dec3, b_dec3):
    grid = (-(-N_NODES // _TC_R),)
    fixed = lambda i: (0, 0)
    enc, dec = pl.pallas_call(
        _tc_dense_body,
        grid=grid,
        in_specs=[
            pl.BlockSpec((_TC_R, D_FEAT), lambda i: (i, 0)),
            pl.BlockSpec((NC, _TC_R, D_FEAT), lambda i: (0, i, 0)),
            pl.BlockSpec((IN_DIM, H2), fixed),
            pl.BlockSpec((1, H2), fixed),
            pl.BlockSpec((H2, EMB), fixed),
            pl.BlockSpec((1, EMB), fixed),
            pl.BlockSpec((EMB, H2), fixed),
            pl.BlockSpec((1, H2), fixed),
            pl.BlockSpec((H2, IN_DIM), fixed),
            pl.BlockSpec((1, IN_DIM), fixed),
        ],
        out_specs=[
            pl.BlockSpec((_TC_R, EMB), lambda i: (i, 0)),
            pl.BlockSpec((_TC_R, IN_DIM), lambda i: (i, 0)),
        ],
        out_shape=[
            jax.ShapeDtypeStruct((N_NODES, EMB), jnp.float32),
            jax.ShapeDtypeStruct((N_NODES, IN_DIM), jnp.float32),
        ],
    )(xp, parts, W_enc1, b_enc1.reshape(1, H2), W_enc3,
      b_enc3.reshape(1, EMB), W_dec1, b_dec1.reshape(1, H2), W_dec3,
      b_dec3.reshape(1, IN_DIM))
    return enc, dec


def kernel(x, edge_index, W_enc1, b_enc1, W_enc3, b_enc3,
           W_dec1, b_dec1, W_dec3, b_dec3):
    epk = edge_index.reshape(2, CH, K)

    parts = _make_sc_agg()(x, epk)

    enc, dec = _tc_dense(x, parts, W_enc1, b_enc1, W_enc3, b_enc3,
                         W_dec1, b_dec1, W_dec3, b_dec3)
    return enc, dec


# speedup vs baseline: 1.2238x; 1.2238x over previous
import functools

import jax
import jax.numpy as jnp
from jax import lax
from jax.experimental import pallas as pl
from jax.experimental.pallas import tpu as pltpu
from jax.experimental.pallas import tpu_sc as plsc

N_NODES = 10000
D_FEAT = 128
ROWS = 10112
N_EDGES = 320000
NC, NS = 2, 16
NW = NC * NS
K = 80
CH = N_EDGES // K
NCH0 = 130
NCH1 = 120
STRIPE = ROWS // NS

IN_DIM = 2 * D_FEAT
H2 = 192
EMB = 128


@functools.cache
def _make_sc_agg():
    mesh = plsc.VectorSubcoreMesh(
        core_axis_name="c", subcore_axis_name="s",
        num_cores=NC, num_subcores=NS)

    @functools.partial(
        pl.kernel,
        out_type=jax.ShapeDtypeStruct((NC, ROWS, D_FEAT), jnp.float32),
        mesh=mesh,
        scratch_types=[
            pltpu.VMEM((NCH0, K), jnp.int32),
            pltpu.VMEM((NCH0, K), jnp.int32),
            pltpu.VMEM((K, D_FEAT), jnp.float32),
            pltpu.VMEM((K, D_FEAT), jnp.float32),
            pltpu.VMEM_SHARED((ROWS, D_FEAT), jnp.float32),
            pltpu.SemaphoreType.DMA,
            pltpu.SemaphoreType.DMA,
        ],
        compiler_params=pltpu.CompilerParams(use_tc_tiling_on_sc=False,
                                             needs_layout_passes=False),
    )
    def sc_agg(x_hbm, epk_hbm, parts_out,
               sidx, didx, buf0, buf1, acc, sem0, sem1):
        c = lax.axis_index("c")
        s = lax.axis_index("s")
        zeros = jnp.zeros((16,), jnp.float32)
        ones = jnp.ones((16,), jnp.float32)
        zeros_i = jnp.zeros((16,), jnp.int32)
        iota16 = lax.broadcasted_iota(jnp.int32, (16,), 0)

        def set_count_col(buf):
            for g in range(K // 16):
                plsc.store_scatter(buf, [iota16 + (g * 16), zeros_i], ones)

        def _zrow(i, _):
            for g in range(D_FEAT // 16):
                buf0[i, pl.ds(g * 16, 16)] = zeros
            return _
        lax.fori_loop(0, K, _zrow, None)

        for kk in range(STRIPE // K):
            pltpu.sync_copy(buf0, acc.at[pl.ds(s * STRIPE + kk * K, K)])
        rem = STRIPE % K
        if rem:
            pltpu.sync_copy(buf0.at[pl.ds(0, rem)],
                            acc.at[pl.ds(s * STRIPE + (STRIPE // K) * K, rem)])
        plsc.subcore_barrier()

        def run_range(row0, nch):
            pltpu.sync_copy(epk_hbm.at[0, pl.ds(row0, nch)],
                            sidx.at[pl.ds(0, nch)])
            pltpu.sync_copy(epk_hbm.at[1, pl.ds(row0, nch)],
                            didx.at[pl.ds(0, nch)])
            pltpu.async_copy(x_hbm.at[sidx.at[0]], buf0, sem0)

            def body(i, _):
                j = 2 * i
                pltpu.async_copy(x_hbm.at[sidx.at[j + 1]], buf1, sem1)
                pltpu.make_async_copy(x_hbm.at[sidx.at[j]], buf0,
                                      sem0).wait()
                set_count_col(buf0)
                pltpu.sync_copy(buf0, acc.at[didx.at[j]], add=True)

                @pl.when(j + 2 < nch)
                def _():
                    pltpu.async_copy(x_hbm.at[sidx.at[j + 2]], buf0, sem0)

                pltpu.make_async_copy(x_hbm.at[sidx.at[j + 1]], buf1,
                                      sem1).wait()
                set_count_col(buf1)
                pltpu.sync_copy(buf1, acc.at[didx.at[j + 1]], add=True)
                return _

            lax.fori_loop(0, nch // 2, body, None)

        @pl.when(c == 0)
        def _():
            run_range(s * NCH0, NCH0)

        @pl.when(c == 1)
        def _():
            run_range(NS * NCH0 + s * NCH1, NCH1)

        plsc.subcore_barrier()
        pltpu.sync_copy(acc.at[pl.ds(s * STRIPE, STRIPE)],
                        parts_out.at[c, pl.ds(s * STRIPE, STRIPE)])

    return sc_agg


def _tc_dense_body(x_ref, parts_ref, w1_ref, b1_ref, w2_ref, b2_ref,
                   w3_ref, b3_ref, w4_ref, b4_ref, enc_ref, dec_ref):
    xs = x_ref[...]
    p = parts_ref[0] + parts_ref[1]
    cnt = p[:, 0:1]
    agg = p / jnp.maximum(cnt, 1.0)
    col = lax.broadcasted_iota(jnp.int32, xs.shape, 1)
    xz = jnp.where(col == 0, 0.0, xs)
    aggz = jnp.where(col == 0, 0.0, agg)
    w1 = w1_ref[...]
    h = jnp.maximum(
        jnp.dot(xz, w1[:D_FEAT], preferred_element_type=jnp.float32)
        + jnp.dot(aggz, w1[D_FEAT:], preferred_element_type=jnp.float32)
        + b1_ref[...], 0.0)
    enc = jnp.dot(h, w2_ref[...], preferred_element_type=jnp.float32) + b2_ref[...]
    enc_ref[...] = enc
    h2 = jnp.maximum(
        jnp.dot(enc, w3_ref[...], preferred_element_type=jnp.float32)
        + b3_ref[...], 0.0)
    dec_ref[...] = (jnp.dot(h2, w4_ref[...], preferred_element_type=jnp.float32)
                    + b4_ref[...])


_TC_R = 2528


def _tc_dense(xp, parts, W_enc1, b_enc1, W_enc3, b_enc3,
              W_dec1, b_dec1, W_dec3, b_dec3):
    grid = (-(-N_NODES // _TC_R),)
    fixed = lambda i: (0, 0)
    enc, dec = pl.pallas_call(
        _tc_dense_body,
        grid=grid,
        in_specs=[
            pl.BlockSpec((_TC_R, D_FEAT), lambda i: (i, 0)),
            pl.BlockSpec((NC, _TC_R, D_FEAT), lambda i: (0, i, 0)),
            pl.BlockSpec((IN_DIM, H2), fixed),
            pl.BlockSpec((1, H2), fixed),
            pl.BlockSpec((H2, EMB), fixed),
            pl.BlockSpec((1, EMB), fixed),
            pl.BlockSpec((EMB, H2), fixed),
            pl.BlockSpec((1, H2), fixed),
            pl.BlockSpec((H2, IN_DIM), fixed),
            pl.BlockSpec((1, IN_DIM), fixed),
        ],
        out_specs=[
            pl.BlockSpec((_TC_R, EMB), lambda i: (i, 0)),
            pl.BlockSpec((_TC_R, IN_DIM), lambda i: (i, 0)),
        ],
        out_shape=[
            jax.ShapeDtypeStruct((N_NODES, EMB), jnp.float32),
            jax.ShapeDtypeStruct((N_NODES, IN_DIM), jnp.float32),
        ],
    )(xp, parts, W_enc1, b_enc1.reshape(1, H2), W_enc3,
      b_enc3.reshape(1, EMB), W_dec1, b_dec1.reshape(1, H2), W_dec3,
      b_dec3.reshape(1, IN_DIM))
    return enc, dec


def kernel(x, edge_index, W_enc1, b_enc1, W_enc3, b_enc3,
           W_dec1, b_dec1, W_dec3, b_dec3):
    epk = edge_index.reshape(2, CH, K)

    parts = _make_sc_agg()(x, epk)

    enc, dec = _tc_dense(x, parts, W_enc1, b_enc1, W_enc3, b_enc3,
                         W_dec1, b_dec1, W_dec3, b_dec3)
    return enc, dec


# speedup vs baseline: 1.2477x; 1.0195x over previous
import functools

import jax
import jax.numpy as jnp
from jax import lax
from jax.experimental import pallas as pl
from jax.experimental.pallas import tpu as pltpu
from jax.experimental.pallas import tpu_sc as plsc

N_NODES = 10000
D_FEAT = 128
ROWS = 10112
N_EDGES = 320000
NC, NS = 2, 16
NW = NC * NS
K = 80
CH = N_EDGES // K
NCH0 = 126
NCH1 = 124
STRIPE = ROWS // NS

IN_DIM = 2 * D_FEAT
H2 = 192
EMB = 128


@functools.cache
def _make_sc_agg():
    mesh = plsc.VectorSubcoreMesh(
        core_axis_name="c", subcore_axis_name="s",
        num_cores=NC, num_subcores=NS)

    @functools.partial(
        pl.kernel,
        out_type=jax.ShapeDtypeStruct((NC, ROWS, D_FEAT), jnp.float32),
        mesh=mesh,
        scratch_types=[
            pltpu.VMEM((NCH0, K), jnp.int32),
            pltpu.VMEM((NCH0, K), jnp.int32),
            pltpu.VMEM((K, D_FEAT), jnp.float32),
            pltpu.VMEM((K, D_FEAT), jnp.float32),
            pltpu.VMEM_SHARED((ROWS, D_FEAT), jnp.float32),
            pltpu.SemaphoreType.DMA,
            pltpu.SemaphoreType.DMA,
        ],
        compiler_params=pltpu.CompilerParams(use_tc_tiling_on_sc=False,
                                             needs_layout_passes=False),
    )
    def sc_agg(x_hbm, epk_hbm, parts_out,
               sidx, didx, buf0, buf1, acc, sem0, sem1):
        c = lax.axis_index("c")
        s = lax.axis_index("s")
        zeros = jnp.zeros((16,), jnp.float32)
        ones = jnp.ones((16,), jnp.float32)
        zeros_i = jnp.zeros((16,), jnp.int32)
        iota16 = lax.broadcasted_iota(jnp.int32, (16,), 0)

        def set_count_col(buf):
            for g in range(K // 16):
                plsc.store_scatter(buf, [iota16 + (g * 16), zeros_i], ones)

        def _zrow(i, _):
            for g in range(D_FEAT // 16):
                buf0[i, pl.ds(g * 16, 16)] = zeros
            return _
        lax.fori_loop(0, K, _zrow, None)

        for kk in range(STRIPE // K):
            pltpu.sync_copy(buf0, acc.at[pl.ds(s * STRIPE + kk * K, K)])
        rem = STRIPE % K
        if rem:
            pltpu.sync_copy(buf0.at[pl.ds(0, rem)],
                            acc.at[pl.ds(s * STRIPE + (STRIPE // K) * K, rem)])

        def run_range(row0, nch):
            pltpu.sync_copy(epk_hbm.at[0, pl.ds(row0, nch)],
                            sidx.at[pl.ds(0, nch)])
            pltpu.sync_copy(epk_hbm.at[1, pl.ds(row0, nch)],
                            didx.at[pl.ds(0, nch)])
            pltpu.async_copy(x_hbm.at[sidx.at[0]], buf0, sem0)
            plsc.subcore_barrier()

            def body(i, _):
                j = 2 * i
                pltpu.async_copy(x_hbm.at[sidx.at[j + 1]], buf1, sem1)
                pltpu.make_async_copy(x_hbm.at[sidx.at[j]], buf0,
                                      sem0).wait()
                set_count_col(buf0)
                pltpu.sync_copy(buf0, acc.at[didx.at[j]], add=True)

                @pl.when(j + 2 < nch)
                def _():
                    pltpu.async_copy(x_hbm.at[sidx.at[j + 2]], buf0, sem0)

                pltpu.make_async_copy(x_hbm.at[sidx.at[j + 1]], buf1,
                                      sem1).wait()
                set_count_col(buf1)
                pltpu.sync_copy(buf1, acc.at[didx.at[j + 1]], add=True)
                return _

            lax.fori_loop(0, nch // 2, body, None)

        @pl.when(c == 0)
        def _():
            run_range(s * NCH0, NCH0)

        @pl.when(c == 1)
        def _():
            run_range(NS * NCH0 + s * NCH1, NCH1)

        plsc.subcore_barrier()
        pltpu.sync_copy(acc.at[pl.ds(s * STRIPE, STRIPE)],
                        parts_out.at[c, pl.ds(s * STRIPE, STRIPE)])

    return sc_agg


def _tc_dense_body(x_ref, parts_ref, w1_ref, b1_ref, w2_ref, b2_ref,
                   w3_ref, b3_ref, w4_ref, b4_ref, enc_ref, dec_ref):
    xs = x_ref[...]
    p = parts_ref[0] + parts_ref[1]
    cnt = p[:, 0:1]
    agg = p / jnp.maximum(cnt, 1.0)
    col = lax.broadcasted_iota(jnp.int32, xs.shape, 1)
    xz = jnp.where(col == 0, 0.0, xs)
    aggz = jnp.where(col == 0, 0.0, agg)
    w1 = w1_ref[...]
    h = jnp.maximum(
        jnp.dot(xz, w1[:D_FEAT], preferred_element_type=jnp.float32)
        + jnp.dot(aggz, w1[D_FEAT:], preferred_element_type=jnp.float32)
        + b1_ref[...], 0.0)
    enc = jnp.dot(h, w2_ref[...], preferred_element_type=jnp.float32) + b2_ref[...]
    enc_ref[...] = enc
    h2 = jnp.maximum(
        jnp.dot(enc, w3_ref[...], preferred_element_type=jnp.float32)
        + b3_ref[...], 0.0)
    dec_ref[...] = (jnp.dot(h2, w4_ref[...], preferred_element_type=jnp.float32)
                    + b4_ref[...])


_TC_R = 2528


def _tc_dense(xp, parts, W_enc1, b_enc1, W_enc3, b_enc3,
              W_dec1, b_dec1, W_dec3, b_dec3):
    grid = (-(-N_NODES // _TC_R),)
    fixed = lambda i: (0, 0)
    enc, dec = pl.pallas_call(
        _tc_dense_body,
        grid=grid,
        in_specs=[
            pl.BlockSpec((_TC_R, D_FEAT), lambda i: (i, 0)),
            pl.BlockSpec((NC, _TC_R, D_FEAT), lambda i: (0, i, 0)),
            pl.BlockSpec((IN_DIM, H2), fixed),
            pl.BlockSpec((1, H2), fixed),
            pl.BlockSpec((H2, EMB), fixed),
            pl.BlockSpec((1, EMB), fixed),
            pl.BlockSpec((EMB, H2), fixed),
            pl.BlockSpec((1, H2), fixed),
            pl.BlockSpec((H2, IN_DIM), fixed),
            pl.BlockSpec((1, IN_DIM), fixed),
        ],
        out_specs=[
            pl.BlockSpec((_TC_R, EMB), lambda i: (i, 0)),
            pl.BlockSpec((_TC_R, IN_DIM), lambda i: (i, 0)),
        ],
        out_shape=[
            jax.ShapeDtypeStruct((N_NODES, EMB), jnp.float32),
            jax.ShapeDtypeStruct((N_NODES, IN_DIM), jnp.float32),
        ],
    )(xp, parts, W_enc1, b_enc1.reshape(1, H2), W_enc3,
      b_enc3.reshape(1, EMB), W_dec1, b_dec1.reshape(1, H2), W_dec3,
      b_dec3.reshape(1, IN_DIM))
    return enc, dec


def kernel(x, edge_index, W_enc1, b_enc1, W_enc3, b_enc3,
           W_dec1, b_dec1, W_dec3, b_dec3):
    epk = edge_index.reshape(2, CH, K)

    parts = _make_sc_agg()(x, epk)

    enc, dec = _tc_dense(x, parts, W_enc1, b_enc1, W_enc3, b_enc3,
                         W_dec1, b_dec1, W_dec3, b_dec3)
    return enc, dec
